# baseline (speedup 1.0000x reference)
import jax
import jax.numpy as jnp
from jax.experimental import pallas as pl
from jax.experimental.pallas import tpu as pltpu

_B, _A, _L = 128, 32, 8192


def _diag_sum(p):
    r = jax.lax.broadcasted_iota(jnp.int32, p.shape, 0)
    c = jax.lax.broadcasted_iota(jnp.int32, p.shape, 1)
    return jnp.sum(jnp.where(r == c, p, 0.0))


_CPB = 8


def _mh_body(pos_ref, u_ref, theta_ref, g_ref, w_ref,
             out_ref, e_ref, acc_ref):
    for c in range(_CPB):
        _mh_chain(c, pos_ref, u_ref, theta_ref, g_ref, w_ref,
                  out_ref, e_ref, acc_ref)


def _mh_chain(c, pos_ref, u_ref, theta_ref, g_ref, w_ref,
              out_ref, e_ref, acc_ref):
    b = pl.program_id(0) * _CPB + c
    t = theta_ref[c]
    w = w_ref[...]
    gt = g_ref[c]

    pos_b = pos_ref[b]
    lane = jax.lax.broadcasted_iota(jnp.int32, t.shape, 1)
    arow = jax.lax.broadcasted_iota(jnp.int32, t.shape, 0)

    s = t * 115.0
    score = jnp.where(lane == pos_b, -s, s) + gt

    m = jnp.max(score, axis=0, keepdims=True)
    idx = jnp.min(jnp.where(score == m, arow, _A), axis=0, keepdims=True)
    newp = jnp.where(arow == idx, 1.0, 0.0).astype(t.dtype)

    dn = (((1,), (1,)), ((), ()))
    tb = t.astype(jnp.bfloat16)
    npb = newp.astype(jnp.bfloat16)
    old_e = _diag_sum(jax.lax.dot_general(
        tb, w, dn, preferred_element_type=jnp.float32))
    new_e = _diag_sum(jax.lax.dot_general(
        npb, w, dn, preferred_element_type=jnp.float32))
    acc = u_ref[b] <= (old_e - new_e)

    out_ref[c] = jnp.where(acc, newp, t)
    e_ref[b] = jnp.where(acc, new_e, old_e)
    acc_ref[b] = jnp.where(acc, 1, 0)


def kernel(theta, W):
    B, A, L = theta.shape
    kr = jax.random.key(42)
    k_pos, k_gumbel, k_u = jax.random.split(kr, 3)

    pos = jnp.argmin(jax.random.uniform(k_pos, (B, L)), axis=-1)
    pos = pos.astype(jnp.int32)
    g = jnp.swapaxes(jax.random.gumbel(k_gumbel, (B, L, A), dtype=theta.dtype),
                     1, 2)
    u = jnp.log(jax.random.uniform(k_u, (B,), dtype=theta.dtype))
    Wq = W.astype(jnp.bfloat16)

    sample, energy, accept = pl.pallas_call(
        _mh_body,
        grid=(B // _CPB,),
        in_specs=[
            pl.BlockSpec(memory_space=pltpu.SMEM),
            pl.BlockSpec(memory_space=pltpu.SMEM),
            pl.BlockSpec((_CPB, A, L), lambda b: (b, 0, 0)),
            pl.BlockSpec((_CPB, A, L), lambda b: (b, 0, 0)),
            pl.BlockSpec((A, L), lambda b: (0, 0)),
        ],
        out_specs=[
            pl.BlockSpec((_CPB, A, L), lambda b: (b, 0, 0)),
            pl.BlockSpec(memory_space=pltpu.SMEM),
            pl.BlockSpec(memory_space=pltpu.SMEM),
        ],
        out_shape=[
            jax.ShapeDtypeStruct((B, A, L), theta.dtype),
            jax.ShapeDtypeStruct((B,), theta.dtype),
            jax.ShapeDtypeStruct((B,), jnp.int32),
        ],
    )(pos, u, theta, g, Wq)

    return sample, energy, accept.astype(bool)


# speedup vs baseline: 1.0118x; 1.0118x over previous
import jax
import jax.numpy as jnp
from jax.experimental import pallas as pl
from jax.experimental.pallas import tpu as pltpu

_B, _A, _L = 128, 32, 8192


def _diag_sum(p):
    r = jax.lax.broadcasted_iota(jnp.int32, p.shape, 0)
    c = jax.lax.broadcasted_iota(jnp.int32, p.shape, 1)
    return jnp.sum(jnp.where(r == c, p, 0.0))


_CPB = 4


def _mh_body(pos_ref, u_ref, theta_ref, g_ref, w_ref,
             out_ref, e_ref, acc_ref):
    for c in range(_CPB):
        _mh_chain(c, pos_ref, u_ref, theta_ref, g_ref, w_ref,
                  out_ref, e_ref, acc_ref)


def _mh_chain(c, pos_ref, u_ref, theta_ref, g_ref, w_ref,
              out_ref, e_ref, acc_ref):
    b = pl.program_id(0) * _CPB + c
    t = theta_ref[c]
    w = w_ref[...]
    gt = g_ref[c]

    pos_b = pos_ref[b]
    lane = jax.lax.broadcasted_iota(jnp.int32, t.shape, 1)
    arow = jax.lax.broadcasted_iota(jnp.int32, t.shape, 0)

    s = t * 115.0
    score = jnp.where(lane == pos_b, -s, s) + gt

    m = jnp.max(score, axis=0, keepdims=True)
    idx = jnp.min(jnp.where(score == m, arow, _A), axis=0, keepdims=True)
    newp = jnp.where(arow == idx, 1.0, 0.0).astype(t.dtype)

    dn = (((1,), (1,)), ((), ()))
    tb = t.astype(jnp.bfloat16)
    npb = newp.astype(jnp.bfloat16)
    old_e = _diag_sum(jax.lax.dot_general(
        tb, w, dn, preferred_element_type=jnp.float32))
    new_e = _diag_sum(jax.lax.dot_general(
        npb, w, dn, preferred_element_type=jnp.float32))
    acc = u_ref[b] <= (old_e - new_e)

    out_ref[c] = jnp.where(acc, newp, t)
    e_ref[b] = jnp.where(acc, new_e, old_e)
    acc_ref[b] = jnp.where(acc, 1, 0)


def kernel(theta, W):
    B, A, L = theta.shape
    kr = jax.random.key(42)
    k_pos, k_gumbel, k_u = jax.random.split(kr, 3)

    pos = jnp.argmin(jax.random.uniform(k_pos, (B, L)), axis=-1)
    pos = pos.astype(jnp.int32)
    g = jnp.swapaxes(jax.random.gumbel(k_gumbel, (B, L, A), dtype=theta.dtype),
                     1, 2)
    u = jnp.log(jax.random.uniform(k_u, (B,), dtype=theta.dtype))
    Wq = W.astype(jnp.bfloat16)

    sample, energy, accept = pl.pallas_call(
        _mh_body,
        grid=(B // _CPB,),
        in_specs=[
            pl.BlockSpec(memory_space=pltpu.SMEM),
            pl.BlockSpec(memory_space=pltpu.SMEM),
            pl.BlockSpec((_CPB, A, L), lambda b: (b, 0, 0)),
            pl.BlockSpec((_CPB, A, L), lambda b: (b, 0, 0)),
            pl.BlockSpec((A, L), lambda b: (0, 0)),
        ],
        out_specs=[
            pl.BlockSpec((_CPB, A, L), lambda b: (b, 0, 0)),
            pl.BlockSpec(memory_space=pltpu.SMEM),
            pl.BlockSpec(memory_space=pltpu.SMEM),
        ],
        out_shape=[
            jax.ShapeDtypeStruct((B, A, L), theta.dtype),
            jax.ShapeDtypeStruct((B,), theta.dtype),
            jax.ShapeDtypeStruct((B,), jnp.int32),
        ],
    )(pos, u, theta, g, Wq)

    return sample, energy, accept.astype(bool)
